# baseline (speedup 1.0000x reference)
import functools

import jax
import jax.numpy as jnp
from jax import lax
from jax.experimental import pallas as pl
from jax.experimental.pallas import tpu as pltpu
from jax.experimental.pallas import tpu_sc as plsc

_N_E = 8192
_D = 32
_BETA = 0.25
_TOK = 8192
_TOK_TILE = 1024
_CODE_TILE = 512
_ACC_CHUNK = 2048
_N_CODE_TILES = _N_E // _CODE_TILE
_GRID = _TOK // _TOK_TILE

_NW = 32
_BPW = _TOK // _NW


def _dist_argmin_body(zb_ref, zsq_ref, esq_ref, e2_ref, idx_ref, loss_ref):
    z = zb_ref[...].astype(jnp.float32)
    zsq = zsq_ref[...]
    acc_v = None
    acc_e = None
    acc_i = None
    for cc in range(_N_E // _ACC_CHUNK):
        best_v = None
        best_i = None
        for c in range(_ACC_CHUNK // _CODE_TILE):
            lo = cc * _ACC_CHUNK + c * _CODE_TILE
            e2 = e2_ref[lo:lo + _CODE_TILE, :]
            m2 = lax.dot_general(z, e2, (((1,), (1,)), ((), ())),
                                 preferred_element_type=jnp.float32)
            esq = esq_ref[0:1, lo:lo + _CODE_TILE]
            d = (zsq + esq) - m2
            lvk = jnp.min(d, axis=1, keepdims=True)
            iota = lax.broadcasted_iota(jnp.int32, (_TOK_TILE, _CODE_TILE), 1)
            li = jnp.min(jnp.where(d == lvk, iota, _N_E), axis=1) + lo
            lv = lvk.reshape(_TOK_TILE)
            if best_v is None:
                best_v, best_i = lv, li
            else:
                upd = lv < best_v
                best_v = jnp.where(upd, lv, best_v)
                best_i = jnp.where(upd, li, best_i)
        if acc_v is None:
            acc_e, acc_i = best_v, best_i
            acc_v = best_v.astype(jnp.bfloat16).astype(jnp.float32)
        else:
            take = best_v < acc_v
            acc_e = jnp.where(take, best_v, acc_e)
            acc_i = jnp.where(take, best_i, acc_i)
            acc_v = jnp.where(take, best_v, acc_v).astype(
                jnp.bfloat16).astype(jnp.float32)
    idx_ref[...] = acc_i.reshape(1, 1, _TOK_TILE)
    s = jnp.sum(acc_e)

    @pl.when(pl.program_id(0) == 0)
    def _init():
        loss_ref[0, 0] = s

    @pl.when(pl.program_id(0) != 0)
    def _acc():
        loss_ref[0, 0] = loss_ref[0, 0] + s


def _dist_argmin(zb, zsq, esq, e2):
    return pl.pallas_call(
        _dist_argmin_body,
        grid=(_GRID,),
        in_specs=[
            pl.BlockSpec((_TOK_TILE, _D), lambda i: (i, 0)),
            pl.BlockSpec((_TOK_TILE, 1), lambda i: (i, 0)),
            pl.BlockSpec((1, _N_E), lambda i: (0, 0)),
            pl.BlockSpec((_N_E, _D), lambda i: (0, 0)),
        ],
        out_specs=[
            pl.BlockSpec((1, 1, _TOK_TILE), lambda i: (i, 0, 0)),
            pl.BlockSpec(memory_space=pltpu.SMEM),
        ],
        out_shape=[
            jax.ShapeDtypeStruct((_GRID, 1, _TOK_TILE), jnp.int32),
            jax.ShapeDtypeStruct((1, 1), jnp.float32),
        ],
    )(zb, zsq, esq, e2)


def _sc_gather_body(table_hbm, idx_hbm, out_hbm, idx_v, rows_v, sem):
    wid = lax.axis_index("s") * 2 + lax.axis_index("c")
    base = wid * _BPW
    pltpu.sync_copy(idx_hbm.at[pl.ds(base, _BPW)], idx_v)
    pltpu.async_copy(table_hbm.at[idx_v], rows_v, sem).wait()
    pltpu.sync_copy(rows_v, out_hbm.at[pl.ds(base, _BPW)])


def _sc_gather(table, idx):
    mesh = plsc.VectorSubcoreMesh(core_axis_name="c", subcore_axis_name="s")
    k = pl.kernel(
        _sc_gather_body,
        out_type=jax.ShapeDtypeStruct((_TOK, _D), jnp.float32),
        mesh=mesh,
        scratch_types=[
            pltpu.VMEM((_BPW,), jnp.int32),
            pltpu.VMEM((_BPW, _D), jnp.float32),
            pltpu.SemaphoreType.DMA,
        ],
        compiler_params=pltpu.CompilerParams(use_tc_tiling_on_sc=False),
    )
    return k(table, idx)


def kernel(z, embedding_weight):
    z_p = jnp.transpose(z, (0, 2, 3, 1))
    z_flat = z_p.reshape(-1, _D)
    zsq = jnp.sum(z_flat ** 2, axis=1, keepdims=True)
    esq = jnp.sum(embedding_weight ** 2, axis=1)[None, :]
    zb = z_flat.astype(jnp.bfloat16)
    e2 = embedding_weight * 2.0

    idx3, loss_sum = _dist_argmin(zb, zsq, esq, e2)
    idx = idx3.reshape(_TOK)
    zq_flat = _sc_gather(embedding_weight, idx)

    m = loss_sum[0, 0] * (1.0 / float(z.size))
    loss = m + _BETA * m

    zq = zq_flat.reshape(z_p.shape)
    z_q = z_p + (zq - z_p)
    z_q = jnp.transpose(z_q, (0, 3, 1, 2))
    return (z_q, loss, idx)


# speedup vs baseline: 1.0987x; 1.0987x over previous
import functools

import jax
import jax.numpy as jnp
from jax import lax
from jax.experimental import pallas as pl
from jax.experimental.pallas import tpu as pltpu
from jax.experimental.pallas import tpu_sc as plsc

_N_E = 8192
_D = 32
_BETA = 0.25
_TOK = 8192
_TOK_TILE = 1024
_CODE_TILE = 512
_ACC_CHUNK = 2048
_N_CODE_TILES = _N_E // _CODE_TILE
_GRID = _TOK // _TOK_TILE

_NW = 32
_BPW = _TOK // _NW


def _dist_argmin_body(zb_ref, zsq_ref, esq_ref, e2_ref, idx_ref, loss_ref):
    z = zb_ref[...].astype(jnp.float32)
    zsq = zsq_ref[...]
    zsq_bits = lax.bitcast_convert_type(zsq, jnp.int32)
    zsq_bits_row = zsq_bits.reshape(_TOK_TILE)
    iota = lax.broadcasted_iota(jnp.int32, (_TOK_TILE, _CODE_TILE), 1)
    acc_v = None
    acc_e = None
    acc_i = None
    for cc in range(_N_E // _ACC_CHUNK):
        best_v = None
        best_i = None
        for c in range(_ACC_CHUNK // _CODE_TILE):
            lo = cc * _ACC_CHUNK + c * _CODE_TILE
            e2 = e2_ref[lo:lo + _CODE_TILE, :]
            m2 = lax.dot_general(z, e2, (((1,), (1,)), ((), ())),
                                 preferred_element_type=jnp.float32)
            esq = esq_ref[0:1, lo:lo + _CODE_TILE]
            d = (zsq + esq) - m2
            delta = lax.bitcast_convert_type(d, jnp.int32) - zsq_bits
            key = ((delta << 9) | iota).astype(jnp.float32)
            k = jnp.min(key, axis=1).astype(jnp.int32)
            li = (k & 511) + lo
            lv = lax.bitcast_convert_type((k >> 9) + zsq_bits_row,
                                          jnp.float32)
            if best_v is None:
                best_v, best_i = lv, li
            else:
                upd = lv < best_v
                best_v = jnp.where(upd, lv, best_v)
                best_i = jnp.where(upd, li, best_i)
        if acc_v is None:
            acc_e, acc_i = best_v, best_i
            acc_v = best_v.astype(jnp.bfloat16).astype(jnp.float32)
        else:
            take = best_v < acc_v
            acc_e = jnp.where(take, best_v, acc_e)
            acc_i = jnp.where(take, best_i, acc_i)
            acc_v = jnp.where(take, best_v, acc_v).astype(
                jnp.bfloat16).astype(jnp.float32)
    idx_ref[...] = acc_i.reshape(1, 1, _TOK_TILE)
    s = jnp.sum(acc_e)

    @pl.when(pl.program_id(0) == 0)
    def _init():
        loss_ref[0, 0] = s

    @pl.when(pl.program_id(0) != 0)
    def _acc():
        loss_ref[0, 0] = loss_ref[0, 0] + s


def _dist_argmin(zb, zsq, esq, e2):
    return pl.pallas_call(
        _dist_argmin_body,
        grid=(_GRID,),
        in_specs=[
            pl.BlockSpec((_TOK_TILE, _D), lambda i: (i, 0)),
            pl.BlockSpec((_TOK_TILE, 1), lambda i: (i, 0)),
            pl.BlockSpec((1, _N_E), lambda i: (0, 0)),
            pl.BlockSpec((_N_E, _D), lambda i: (0, 0)),
        ],
        out_specs=[
            pl.BlockSpec((1, 1, _TOK_TILE), lambda i: (i, 0, 0)),
            pl.BlockSpec(memory_space=pltpu.SMEM),
        ],
        out_shape=[
            jax.ShapeDtypeStruct((_GRID, 1, _TOK_TILE), jnp.int32),
            jax.ShapeDtypeStruct((1, 1), jnp.float32),
        ],
    )(zb, zsq, esq, e2)


def _sc_gather_body(table_hbm, idx_hbm, out_hbm, idx_v, rows_v, sem):
    wid = lax.axis_index("s") * 2 + lax.axis_index("c")
    base = wid * _BPW
    pltpu.sync_copy(idx_hbm.at[pl.ds(base, _BPW)], idx_v)
    pltpu.async_copy(table_hbm.at[idx_v], rows_v, sem).wait()
    pltpu.sync_copy(rows_v, out_hbm.at[pl.ds(base, _BPW)])


def _sc_gather(table, idx):
    mesh = plsc.VectorSubcoreMesh(core_axis_name="c", subcore_axis_name="s")
    k = pl.kernel(
        _sc_gather_body,
        out_type=jax.ShapeDtypeStruct((_TOK, _D), jnp.float32),
        mesh=mesh,
        scratch_types=[
            pltpu.VMEM((_BPW,), jnp.int32),
            pltpu.VMEM((_BPW, _D), jnp.float32),
            pltpu.SemaphoreType.DMA,
        ],
        compiler_params=pltpu.CompilerParams(use_tc_tiling_on_sc=False),
    )
    return k(table, idx)


def kernel(z, embedding_weight):
    z_p = jnp.transpose(z, (0, 2, 3, 1))
    z_flat = z_p.reshape(-1, _D)
    zsq = jnp.sum(z_flat ** 2, axis=1, keepdims=True)
    esq = jnp.sum(embedding_weight ** 2, axis=1)[None, :]
    zb = z_flat.astype(jnp.bfloat16)
    e2 = embedding_weight * 2.0

    idx3, loss_sum = _dist_argmin(zb, zsq, esq, e2)
    idx = idx3.reshape(_TOK)
    zq_flat = _sc_gather(embedding_weight, idx)

    m = loss_sum[0, 0] * (1.0 / float(z.size))
    loss = m + _BETA * m

    zq = zq_flat.reshape(z_p.shape)
    z_q = z_p + (zq - z_p)
    z_q = jnp.transpose(z_q, (0, 3, 1, 2))
    return (z_q, loss, idx)


# speedup vs baseline: 1.1453x; 1.0424x over previous
import functools

import jax
import jax.numpy as jnp
from jax import lax
from jax.experimental import pallas as pl
from jax.experimental.pallas import tpu as pltpu
from jax.experimental.pallas import tpu_sc as plsc

_N_E = 8192
_D = 32
_BETA = 0.25
_TOK = 8192
_TOK_TILE = 1024
_CODE_TILE = 512
_ACC_CHUNK = 2048
_N_CODE_TILES = _N_E // _CODE_TILE
_GRID = _TOK // _TOK_TILE

_NW = 32
_BPW = _TOK // _NW


def _dist_argmin_body(z_ref, zsq_ref, esq_ref, e_ref, idx_ref, loss_ref):
    z = z_ref[...].astype(jnp.bfloat16).astype(jnp.float32) * 2.0
    zsq = zsq_ref[...]
    zsq_bits = lax.bitcast_convert_type(zsq, jnp.int32)
    zsq_bits_row = zsq_bits.reshape(_TOK_TILE)
    iota = lax.broadcasted_iota(jnp.int32, (_TOK_TILE, _CODE_TILE), 1)
    acc_v = None
    acc_e = None
    acc_i = None
    for cc in range(_N_E // _ACC_CHUNK):
        best_v = None
        best_i = None
        for c in range(_ACC_CHUNK // _CODE_TILE):
            lo = cc * _ACC_CHUNK + c * _CODE_TILE
            ec = e_ref[lo:lo + _CODE_TILE, :]
            m2 = lax.dot_general(z, ec, (((1,), (1,)), ((), ())),
                                 preferred_element_type=jnp.float32)
            esq = esq_ref[0:1, lo:lo + _CODE_TILE]
            d = (zsq + esq) - m2
            delta = lax.bitcast_convert_type(d, jnp.int32) - zsq_bits
            key = ((delta << 9) | iota).astype(jnp.float32)
            k = jnp.min(key, axis=1).astype(jnp.int32)
            li = (k & 511) + lo
            lv = lax.bitcast_convert_type((k >> 9) + zsq_bits_row,
                                          jnp.float32)
            if best_v is None:
                best_v, best_i = lv, li
            else:
                upd = lv < best_v
                best_v = jnp.where(upd, lv, best_v)
                best_i = jnp.where(upd, li, best_i)
        if acc_v is None:
            acc_e, acc_i = best_v, best_i
            acc_v = best_v.astype(jnp.bfloat16).astype(jnp.float32)
        else:
            take = best_v < acc_v
            acc_e = jnp.where(take, best_v, acc_e)
            acc_i = jnp.where(take, best_i, acc_i)
            acc_v = jnp.where(take, best_v, acc_v).astype(
                jnp.bfloat16).astype(jnp.float32)
    idx_ref[...] = acc_i.reshape(1, 1, _TOK_TILE)
    s = jnp.sum(acc_e)

    @pl.when(pl.program_id(0) == 0)
    def _init():
        loss_ref[0, 0] = s

    @pl.when(pl.program_id(0) != 0)
    def _acc():
        loss_ref[0, 0] = loss_ref[0, 0] + s


def _dist_argmin(z_flat, zsq, esq, e):
    return pl.pallas_call(
        _dist_argmin_body,
        grid=(_GRID,),
        in_specs=[
            pl.BlockSpec((_TOK_TILE, _D), lambda i: (i, 0)),
            pl.BlockSpec((_TOK_TILE, 1), lambda i: (i, 0)),
            pl.BlockSpec((1, _N_E), lambda i: (0, 0)),
            pl.BlockSpec((_N_E, _D), lambda i: (0, 0)),
        ],
        out_specs=[
            pl.BlockSpec((1, 1, _TOK_TILE), lambda i: (i, 0, 0)),
            pl.BlockSpec(memory_space=pltpu.SMEM),
        ],
        out_shape=[
            jax.ShapeDtypeStruct((_GRID, 1, _TOK_TILE), jnp.int32),
            jax.ShapeDtypeStruct((1, 1), jnp.float32),
        ],
    )(z_flat, zsq, esq, e)


def _sc_gather_body(table_hbm, idx_hbm, out_hbm, idx_v, rows_v, sem):
    wid = lax.axis_index("s") * 2 + lax.axis_index("c")
    base = wid * _BPW
    pltpu.sync_copy(idx_hbm.at[pl.ds(base, _BPW)], idx_v)
    pltpu.async_copy(table_hbm.at[idx_v], rows_v, sem).wait()
    pltpu.sync_copy(rows_v, out_hbm.at[pl.ds(base, _BPW)])


def _sc_gather(table, idx):
    mesh = plsc.VectorSubcoreMesh(core_axis_name="c", subcore_axis_name="s")
    k = pl.kernel(
        _sc_gather_body,
        out_type=jax.ShapeDtypeStruct((_TOK, _D), jnp.float32),
        mesh=mesh,
        scratch_types=[
            pltpu.VMEM((_BPW,), jnp.int32),
            pltpu.VMEM((_BPW, _D), jnp.float32),
            pltpu.SemaphoreType.DMA,
        ],
        compiler_params=pltpu.CompilerParams(use_tc_tiling_on_sc=False),
    )
    return k(table, idx)


def kernel(z, embedding_weight):
    z_p = jnp.transpose(z, (0, 2, 3, 1))
    z_flat = z_p.reshape(-1, _D)
    zsq = jnp.sum(z_flat ** 2, axis=1, keepdims=True)
    esq = jnp.sum(embedding_weight ** 2, axis=1)[None, :]

    idx3, loss_sum = _dist_argmin(z_flat, zsq, esq, embedding_weight)
    idx = idx3.reshape(_TOK)
    zq_flat = _sc_gather(embedding_weight, idx)

    m = loss_sum[0, 0] * (1.0 / float(z.size))
    loss = m + _BETA * m

    zq = zq_flat.reshape(z_p.shape)
    z_q = z_p + (zq - z_p)
    z_q = jnp.transpose(z_q, (0, 3, 1, 2))
    return (z_q, loss, idx)


# speedup vs baseline: 1.4601x; 1.2748x over previous
import functools

import jax
import jax.numpy as jnp
from jax import lax
from jax.experimental import pallas as pl
from jax.experimental.pallas import tpu as pltpu
from jax.experimental.pallas import tpu_sc as plsc

_N_E = 8192
_D = 32
_BETA = 0.25
_TOK = 8192
_TOK_TILE = 1024
_CODE_TILE = 512
_ACC_CHUNK = 2048
_N_CODE_TILES = _N_E // _CODE_TILE
_GRID = _TOK // _TOK_TILE

_NW = 32
_BPW = _TOK // _NW


def _dist_argmin_body(z_ref, zsq_ref, e_ref, idx_ref, loss_ref):
    z = z_ref[...].astype(jnp.bfloat16).astype(jnp.float32) * 2.0
    zsq = zsq_ref[...].reshape(1, _TOK_TILE)
    zsq_bits = lax.bitcast_convert_type(zsq, jnp.int32)
    zsq_bits_row = zsq_bits.reshape(_TOK_TILE)
    iota = lax.broadcasted_iota(jnp.int32, (_CODE_TILE, _TOK_TILE), 0)
    acc_v = None
    acc_e = None
    acc_i = None
    for cc in range(_N_E // _ACC_CHUNK):
        best_v = None
        best_i = None
        for c in range(_ACC_CHUNK // _CODE_TILE):
            lo = cc * _ACC_CHUNK + c * _CODE_TILE
            ec = e_ref[lo:lo + _CODE_TILE, :]
            m2 = lax.dot_general(ec, z, (((1,), (1,)), ((), ())),
                                 preferred_element_type=jnp.float32)
            esq = jnp.sum(ec * ec, axis=1, keepdims=True)
            d = (zsq + esq) - m2
            delta = lax.bitcast_convert_type(d, jnp.int32) - zsq_bits
            key = ((delta << 9) | iota).astype(jnp.float32)
            k = jnp.min(key, axis=0).astype(jnp.int32)
            li = (k & 511) + lo
            lv = lax.bitcast_convert_type((k >> 9) + zsq_bits_row,
                                          jnp.float32)
            if best_v is None:
                best_v, best_i = lv, li
            else:
                upd = lv < best_v
                best_v = jnp.where(upd, lv, best_v)
                best_i = jnp.where(upd, li, best_i)
        if acc_v is None:
            acc_e, acc_i = best_v, best_i
            acc_v = best_v.astype(jnp.bfloat16).astype(jnp.float32)
        else:
            take = best_v < acc_v
            acc_e = jnp.where(take, best_v, acc_e)
            acc_i = jnp.where(take, best_i, acc_i)
            acc_v = jnp.where(take, best_v, acc_v).astype(
                jnp.bfloat16).astype(jnp.float32)
    idx_ref[...] = acc_i.reshape(1, 1, _TOK_TILE)
    s = jnp.sum(acc_e)

    @pl.when(pl.program_id(0) == 0)
    def _init():
        loss_ref[0, 0] = s

    @pl.when(pl.program_id(0) != 0)
    def _acc():
        loss_ref[0, 0] = loss_ref[0, 0] + s


def _dist_argmin(z_flat, zsq2d, e):
    return pl.pallas_call(
        _dist_argmin_body,
        grid=(_GRID,),
        in_specs=[
            pl.BlockSpec((_TOK_TILE, _D), lambda i: (i, 0)),
            pl.BlockSpec((1, 1, _TOK_TILE), lambda i: (i, 0, 0)),
            pl.BlockSpec((_N_E, _D), lambda i: (0, 0)),
        ],
        out_specs=[
            pl.BlockSpec((1, 1, _TOK_TILE), lambda i: (i, 0, 0)),
            pl.BlockSpec(memory_space=pltpu.SMEM),
        ],
        out_shape=[
            jax.ShapeDtypeStruct((_GRID, 1, _TOK_TILE), jnp.int32),
            jax.ShapeDtypeStruct((1, 1), jnp.float32),
        ],
    )(z_flat, zsq2d, e)


def _sc_gather_body(table_hbm, idx_hbm, out_hbm, idx_v, rows_v, sem):
    wid = lax.axis_index("s") * 2 + lax.axis_index("c")
    base = wid * _BPW
    pltpu.sync_copy(idx_hbm.at[pl.ds(base, _BPW)], idx_v)
    pltpu.async_copy(table_hbm.at[idx_v], rows_v, sem).wait()
    pltpu.sync_copy(rows_v, out_hbm.at[pl.ds(base, _BPW)])


def _sc_gather(table, idx):
    mesh = plsc.VectorSubcoreMesh(core_axis_name="c", subcore_axis_name="s")
    k = pl.kernel(
        _sc_gather_body,
        out_type=jax.ShapeDtypeStruct((_TOK, _D), jnp.float32),
        mesh=mesh,
        scratch_types=[
            pltpu.VMEM((_BPW,), jnp.int32),
            pltpu.VMEM((_BPW, _D), jnp.float32),
            pltpu.SemaphoreType.DMA,
        ],
        compiler_params=pltpu.CompilerParams(use_tc_tiling_on_sc=False),
    )
    return k(table, idx)


def kernel(z, embedding_weight):
    z_p = jnp.transpose(z, (0, 2, 3, 1))
    z_flat = z_p.reshape(-1, _D)
    zsq2d = jnp.sum(z_flat ** 2, axis=1).reshape(_GRID, 1, _TOK_TILE)

    idx3, loss_sum = _dist_argmin(z_flat, zsq2d, embedding_weight)
    idx = idx3.reshape(_TOK)
    zq_flat = _sc_gather(embedding_weight, idx)

    m = loss_sum[0, 0] * (1.0 / float(z.size))
    loss = m + _BETA * m

    zq = zq_flat.reshape(z_p.shape)
    z_q = z_p + (zq - z_p)
    z_q = jnp.transpose(z_q, (0, 3, 1, 2))
    return (z_q, loss, idx)


# speedup vs baseline: 1.4735x; 1.0092x over previous
import functools

import jax
import jax.numpy as jnp
from jax import lax
from jax.experimental import pallas as pl
from jax.experimental.pallas import tpu as pltpu
from jax.experimental.pallas import tpu_sc as plsc

_N_E = 8192
_D = 32
_BETA = 0.25
_TOK = 8192
_TOK_TILE = 2048
_CODE_TILE = 512
_ACC_CHUNK = 2048
_N_CODE_TILES = _N_E // _CODE_TILE
_GRID = _TOK // _TOK_TILE

_NW = 32
_BPW = _TOK // _NW


def _dist_argmin_body(z_ref, zsq_ref, e_ref, idx_ref, loss_ref):
    z = z_ref[...].astype(jnp.bfloat16).astype(jnp.float32) * 2.0
    zsq = zsq_ref[...].reshape(1, _TOK_TILE)
    zsq_bits = lax.bitcast_convert_type(zsq, jnp.int32)
    zsq_bits_row = zsq_bits.reshape(_TOK_TILE)
    iota_f = lax.broadcasted_iota(
        jnp.int32, (_CODE_TILE, _TOK_TILE), 0).astype(jnp.float32)
    acc_v = None
    acc_e = None
    acc_i = None
    for cc in range(_N_E // _ACC_CHUNK):
        best_v = None
        best_i = None
        for c in range(_ACC_CHUNK // _CODE_TILE):
            lo = cc * _ACC_CHUNK + c * _CODE_TILE
            ec = e_ref[lo:lo + _CODE_TILE, :]
            m2 = lax.dot_general(ec, z, (((1,), (1,)), ((), ())),
                                 preferred_element_type=jnp.float32)
            esq = jnp.sum(ec * ec, axis=1, keepdims=True)
            d = (zsq + esq) - m2
            delta = lax.bitcast_convert_type(d, jnp.int32) - zsq_bits
            key = delta.astype(jnp.float32) * 512.0 + iota_f
            k = jnp.min(key, axis=0).astype(jnp.int32)
            li = (k & 511) + lo
            lv = lax.bitcast_convert_type((k >> 9) + zsq_bits_row,
                                          jnp.float32)
            if best_v is None:
                best_v, best_i = lv, li
            else:
                upd = lv < best_v
                best_v = jnp.where(upd, lv, best_v)
                best_i = jnp.where(upd, li, best_i)
        if acc_v is None:
            acc_e, acc_i = best_v, best_i
            acc_v = best_v.astype(jnp.bfloat16).astype(jnp.float32)
        else:
            take = best_v < acc_v
            acc_e = jnp.where(take, best_v, acc_e)
            acc_i = jnp.where(take, best_i, acc_i)
            acc_v = jnp.where(take, best_v, acc_v).astype(
                jnp.bfloat16).astype(jnp.float32)
    idx_ref[...] = acc_i.reshape(1, 1, _TOK_TILE)
    s = jnp.sum(acc_e)

    @pl.when(pl.program_id(0) == 0)
    def _init():
        loss_ref[0, 0] = s

    @pl.when(pl.program_id(0) != 0)
    def _acc():
        loss_ref[0, 0] = loss_ref[0, 0] + s


def _dist_argmin(z_flat, zsq2d, e):
    return pl.pallas_call(
        _dist_argmin_body,
        grid=(_GRID,),
        in_specs=[
            pl.BlockSpec((_TOK_TILE, _D), lambda i: (i, 0)),
            pl.BlockSpec((1, 1, _TOK_TILE), lambda i: (i, 0, 0)),
            pl.BlockSpec((_N_E, _D), lambda i: (0, 0)),
        ],
        out_specs=[
            pl.BlockSpec((1, 1, _TOK_TILE), lambda i: (i, 0, 0)),
            pl.BlockSpec(memory_space=pltpu.SMEM),
        ],
        out_shape=[
            jax.ShapeDtypeStruct((_GRID, 1, _TOK_TILE), jnp.int32),
            jax.ShapeDtypeStruct((1, 1), jnp.float32),
        ],
    )(z_flat, zsq2d, e)


def _sc_gather_body(table_hbm, idx_hbm, out_hbm, idx_v, rows_v, sem):
    wid = lax.axis_index("s") * 2 + lax.axis_index("c")
    base = wid * _BPW
    pltpu.sync_copy(idx_hbm.at[pl.ds(base, _BPW)], idx_v)
    pltpu.async_copy(table_hbm.at[idx_v], rows_v, sem).wait()
    pltpu.sync_copy(rows_v, out_hbm.at[pl.ds(base, _BPW)])


def _sc_gather(table, idx):
    mesh = plsc.VectorSubcoreMesh(core_axis_name="c", subcore_axis_name="s")
    k = pl.kernel(
        _sc_gather_body,
        out_type=jax.ShapeDtypeStruct((_TOK, _D), jnp.float32),
        mesh=mesh,
        scratch_types=[
            pltpu.VMEM((_BPW,), jnp.int32),
            pltpu.VMEM((_BPW, _D), jnp.float32),
            pltpu.SemaphoreType.DMA,
        ],
        compiler_params=pltpu.CompilerParams(use_tc_tiling_on_sc=False),
    )
    return k(table, idx)


def kernel(z, embedding_weight):
    z_p = jnp.transpose(z, (0, 2, 3, 1))
    z_flat = z_p.reshape(-1, _D)
    zsq2d = jnp.sum(z_flat ** 2, axis=1).reshape(_GRID, 1, _TOK_TILE)

    idx3, loss_sum = _dist_argmin(z_flat, zsq2d, embedding_weight)
    idx = idx3.reshape(_TOK)
    zq_flat = _sc_gather(embedding_weight, idx)

    m = loss_sum[0, 0] * (1.0 / float(z.size))
    loss = m + _BETA * m

    zq = zq_flat.reshape(z_p.shape)
    z_q = z_p + (zq - z_p)
    z_q = jnp.transpose(z_q, (0, 3, 1, 2))
    return (z_q, loss, idx)


# speedup vs baseline: 1.4795x; 1.0041x over previous
import jax
import jax.numpy as jnp
from jax import lax
from jax.experimental import pallas as pl
from jax.experimental.pallas import tpu as pltpu
from jax.experimental.pallas import tpu_sc as plsc

_N_E = 8192
_D = 32
_BETA = 0.25
_TOK = 8192
_TOK_TILE = 2048
_CODE_TILE = 512
_ACC_CHUNK = 2048
_N_CODE_TILES = _N_E // _CODE_TILE
_GRID = _TOK // _TOK_TILE

_NW = 32
_BPW = _TOK // _NW


def _dist_argmin_body(z_ref, zsq_ref, e_ref, idx_ref, loss_ref):
    z = z_ref[...].astype(jnp.bfloat16).astype(jnp.float32) * 2.0
    zsq = zsq_ref[...].reshape(1, _TOK_TILE)
    zsq_bits = lax.bitcast_convert_type(zsq, jnp.int32)
    zsq_bits_row = zsq_bits.reshape(_TOK_TILE)
    iota_f = lax.broadcasted_iota(
        jnp.int32, (_CODE_TILE, _TOK_TILE), 0).astype(jnp.float32)
    acc_v = None
    acc_e = None
    acc_i = None
    for cc in range(_N_E // _ACC_CHUNK):
        best_v = None
        best_i = None
        for c in range(_ACC_CHUNK // _CODE_TILE):
            lo = cc * _ACC_CHUNK + c * _CODE_TILE
            ec = e_ref[lo:lo + _CODE_TILE, :]
            m2 = lax.dot_general(ec, z, (((1,), (1,)), ((), ())),
                                 preferred_element_type=jnp.float32)
            esq = jnp.sum(ec * ec, axis=1, keepdims=True)
            d = (zsq + esq) - m2
            delta = lax.bitcast_convert_type(d, jnp.int32) - zsq_bits
            key = delta.astype(jnp.float32) * 512.0 + iota_f
            k = jnp.min(key, axis=0).astype(jnp.int32)
            li = (k & 511) + lo
            lv = lax.bitcast_convert_type((k >> 9) + zsq_bits_row,
                                          jnp.float32)
            if best_v is None:
                best_v, best_i = lv, li
            else:
                upd = lv < best_v
                best_v = jnp.where(upd, lv, best_v)
                best_i = jnp.where(upd, li, best_i)
        if acc_v is None:
            acc_e, acc_i = best_v, best_i
            acc_v = best_v.astype(jnp.bfloat16).astype(jnp.float32)
        else:
            take = best_v < acc_v
            acc_e = jnp.where(take, best_v, acc_e)
            acc_i = jnp.where(take, best_i, acc_i)
            acc_v = jnp.where(take, best_v, acc_v).astype(
                jnp.bfloat16).astype(jnp.float32)
    idx_ref[...] = acc_i.reshape(1, 1, _TOK_TILE)
    s = jnp.sum(acc_e)

    @pl.when(pl.program_id(0) == 0)
    def _init():
        loss_ref[0, 0] = s

    @pl.when(pl.program_id(0) != 0)
    def _acc():
        loss_ref[0, 0] = loss_ref[0, 0] + s


def _dist_argmin(z_flat, zsq2d, e):
    return pl.pallas_call(
        _dist_argmin_body,
        grid=(_GRID,),
        in_specs=[
            pl.BlockSpec((_TOK_TILE, _D), lambda i: (i, 0)),
            pl.BlockSpec((1, 1, _TOK_TILE), lambda i: (i, 0, 0)),
            pl.BlockSpec((_N_E, _D), lambda i: (0, 0)),
        ],
        out_specs=[
            pl.BlockSpec((1, 1, _TOK_TILE), lambda i: (i, 0, 0)),
            pl.BlockSpec(memory_space=pltpu.SMEM),
        ],
        out_shape=[
            jax.ShapeDtypeStruct((_GRID, 1, _TOK_TILE), jnp.int32),
            jax.ShapeDtypeStruct((1, 1), jnp.float32),
        ],
    )(z_flat, zsq2d, e)


def _sc_gather_body(table_hbm, idx_hbm, out_hbm, idx_v, rows_v, sem):
    wid = lax.axis_index("s") * 2 + lax.axis_index("c")
    base = wid * _BPW
    pltpu.sync_copy(idx_hbm.at[pl.ds(base, _BPW)], idx_v)
    pltpu.async_copy(table_hbm.at[idx_v], rows_v, sem).wait()
    pltpu.sync_copy(rows_v, out_hbm.at[pl.ds(base, _BPW)])


def _sc_gather(table, idx):
    mesh = plsc.VectorSubcoreMesh(core_axis_name="c", subcore_axis_name="s")
    k = pl.kernel(
        _sc_gather_body,
        out_type=jax.ShapeDtypeStruct((_TOK, _D), jnp.float32),
        mesh=mesh,
        scratch_types=[
            pltpu.VMEM((_BPW,), jnp.int32),
            pltpu.VMEM((_BPW, _D), jnp.float32),
            pltpu.SemaphoreType.DMA,
        ],
        compiler_params=pltpu.CompilerParams(use_tc_tiling_on_sc=False),
    )
    return k(table, idx)


def kernel(z, embedding_weight):
    z_p = jnp.transpose(z, (0, 2, 3, 1))
    z_flat = z_p.reshape(-1, _D)
    zsq2d = jnp.sum(z_flat ** 2, axis=1).reshape(_GRID, 1, _TOK_TILE)

    idx3, loss_sum = _dist_argmin(z_flat, zsq2d, embedding_weight)
    idx = idx3.reshape(_TOK)
    zq_flat = _sc_gather(embedding_weight, idx)

    m = loss_sum[0, 0] * (1.0 / float(z.size))
    loss = m + _BETA * m

    zq = zq_flat.reshape(z_p.shape)
    z_q = z_p + (zq - z_p)
    z_q = jnp.transpose(z_q, (0, 3, 1, 2))
    return (z_q, loss, idx)
